# baseline (speedup 1.0000x reference)
import functools

import jax
import jax.numpy as jnp
from jax import lax
from jax.experimental import pallas as pl
from jax.experimental.pallas import tpu as pltpu


def _qfl_block_t(pred_ref, gts_ref, label_ref, out_ref):
    x = pred_ref[...]
    g = gts_ref[...]
    lab = label_ref[...]
    C, B = x.shape

    rows = lax.broadcasted_iota(jnp.int32, (C, B), 0)
    onehot = rows == lab

    s = 0.5 * jnp.tanh(0.5 * x) + 0.5
    s_abs = jnp.where(x >= 0, s, 1.0 - s)
    sp = -jnp.log(s_abs)
    base = jnp.maximum(x, 0.0) + sp

    a = jnp.where(onehot, g, 0.0)
    t = jnp.where(onehot, g - s, s)
    out_ref[...] = (base - x * a) * (t * t)


@jax.jit
def kernel(pred_logits, gt_label, gt_score):
    N, C = pred_logits.shape
    BN = 28672
    grid = (pl.cdiv(N, BN),)
    pt = pred_logits.T
    gt = gt_score.T
    lab = gt_label.astype(jnp.int32).reshape(1, N)
    out_t = pl.pallas_call(
        _qfl_block_t,
        grid=grid,
        in_specs=[
            pl.BlockSpec((C, BN), lambda i: (0, i)),
            pl.BlockSpec((C, BN), lambda i: (0, i)),
            pl.BlockSpec((1, BN), lambda i: (0, i)),
        ],
        out_specs=pl.BlockSpec((C, BN), lambda i: (0, i)),
        out_shape=jax.ShapeDtypeStruct((C, N), jnp.float32),
    )(pt, gt, lab)
    return out_t.T


# speedup vs baseline: 1.0241x; 1.0241x over previous
import functools

import jax
import jax.numpy as jnp
from jax import lax
from jax.experimental import pallas as pl
from jax.experimental.pallas import tpu as pltpu


def _qfl_block_t(pred_ref, gts_ref, label_ref, out_ref):
    x = pred_ref[...]
    g = gts_ref[...]
    lab = label_ref[...]
    C, B = x.shape

    rows = lax.broadcasted_iota(jnp.int32, (C, B), 0)
    onehot = rows == lab

    s = 0.5 * jnp.tanh(0.5 * x) + 0.5
    s_abs = jnp.where(x >= 0, s, 1.0 - s)
    sp = -jnp.log(s_abs)
    base = jnp.maximum(x, 0.0) + sp

    a = jnp.where(onehot, g, 0.0)
    t = jnp.where(onehot, g - s, s)
    out_ref[...] = (base - x * a) * (t * t)


@jax.jit
def kernel(pred_logits, gt_label, gt_score):
    N, C = pred_logits.shape
    BN = 12544
    grid = (pl.cdiv(N, BN),)
    pt = pred_logits.T
    gt = gt_score.T
    lab = gt_label.astype(jnp.int32).reshape(1, N)
    out_t = pl.pallas_call(
        _qfl_block_t,
        grid=grid,
        in_specs=[
            pl.BlockSpec((C, BN), lambda i: (0, i)),
            pl.BlockSpec((C, BN), lambda i: (0, i)),
            pl.BlockSpec((1, BN), lambda i: (0, i)),
        ],
        out_specs=pl.BlockSpec((C, BN), lambda i: (0, i)),
        out_shape=jax.ShapeDtypeStruct((C, N), jnp.float32),
    )(pt, gt, lab)
    return out_t.T


# speedup vs baseline: 1.0982x; 1.0724x over previous
import functools

import jax
import jax.numpy as jnp
from jax import lax
from jax.experimental import pallas as pl
from jax.experimental.pallas import tpu as pltpu


def _qfl_block_t(pred_ref, gts_ref, label_ref, out_ref):
    x = pred_ref[...]
    g = gts_ref[...]
    lab = label_ref[...]
    C, B = x.shape

    rows = lax.broadcasted_iota(jnp.int32, (C, B), 0)
    onehot = rows == lab

    s = 0.5 * jnp.tanh(0.5 * x) + 0.5
    s_abs = jnp.where(x >= 0, s, 1.0 - s)
    sp = -jnp.log(s_abs)
    base = jnp.maximum(x, 0.0) + sp

    a = jnp.where(onehot, g, 0.0)
    t = jnp.where(onehot, g - s, s)
    out_ref[...] = (base - x * a) * (t * t)


@jax.jit
def kernel(pred_logits, gt_label, gt_score):
    N, C = pred_logits.shape
    BN = 25600
    grid = (pl.cdiv(N, BN),)
    pt = pred_logits.T
    gt = gt_score.T
    lab = gt_label.astype(jnp.int32)
    out_t = pl.pallas_call(
        _qfl_block_t,
        grid=grid,
        in_specs=[
            pl.BlockSpec((C, BN), lambda i: (0, i)),
            pl.BlockSpec((C, BN), lambda i: (0, i)),
            pl.BlockSpec((BN,), lambda i: (i,)),
        ],
        out_specs=pl.BlockSpec((C, BN), lambda i: (0, i)),
        out_shape=jax.ShapeDtypeStruct((C, N), jnp.float32),
    )(pt, gt, lab)
    return out_t.T


# speedup vs baseline: 1.0993x; 1.0010x over previous
import functools

import jax
import jax.numpy as jnp
from jax import lax
from jax.experimental import pallas as pl
from jax.experimental.pallas import tpu as pltpu


def _qfl_block_t(pred_ref, gts_ref, label_ref, out_ref):
    x = pred_ref[...]
    g = gts_ref[...]
    lab = label_ref[...]
    C, B = x.shape

    rows = lax.broadcasted_iota(jnp.int32, (C, B), 0)
    onehot = rows == lab

    s = 0.5 * jnp.tanh(0.5 * x) + 0.5
    s_abs = jnp.where(x >= 0, s, 1.0 - s)
    sp = -jnp.log(s_abs)
    base = jnp.maximum(x, 0.0) + sp

    a = jnp.where(onehot, g, 0.0)
    t = jnp.where(onehot, g - s, s)
    out_ref[...] = (base - x * a) * (t * t)


@jax.jit
def kernel(pred_logits, gt_label, gt_score):
    N, C = pred_logits.shape
    BN = 20480
    grid = (pl.cdiv(N, BN),)
    pt = pred_logits.T
    gt = gt_score.T
    lab = gt_label.astype(jnp.int32)
    out_t = pl.pallas_call(
        _qfl_block_t,
        grid=grid,
        in_specs=[
            pl.BlockSpec((C, BN), lambda i: (0, i)),
            pl.BlockSpec((C, BN), lambda i: (0, i)),
            pl.BlockSpec((BN,), lambda i: (i,)),
        ],
        out_specs=pl.BlockSpec((C, BN), lambda i: (0, i)),
        out_shape=jax.ShapeDtypeStruct((C, N), jnp.float32),
    )(pt, gt, lab)
    return out_t.T
